# baseline (speedup 1.0000x reference)
import functools

import jax
import jax.numpy as jnp
from jax import lax
from jax.experimental import pallas as pl
from jax.experimental.pallas import tpu as pltpu
from jax.experimental.pallas import tpu_sc as plsc

L2_NORM = 0.0001

B = 4096
L = 50
D = 128
NW = 32
BPW = B // NW
CHUNK_U = 2
CHUNK_I = CHUNK_U * L
NCHUNK = BPW // CHUNK_U
DC = D // 16
NBUF = 4


def _sc_kernel(users_hbm, items_hbm, uemb_hbm, iemb_hbm,
               pred_hbm, l2_hbm,
               uidx_v, iidx_v, urows_v,
               irows0_v, irows1_v, irows2_v, irows3_v, predbuf_v,
               l2buf_v, sqi_v, squ_v, sem0, sem1, sem2, sem3, sem_u):
    wid = lax.axis_index("s") * 2 + lax.axis_index("c")
    base = wid * BPW
    bufs = (irows0_v, irows1_v, irows2_v, irows3_v)
    sems = (sem0, sem1, sem2, sem3)

    zero = jnp.zeros((16,), jnp.float32)
    sqi_v[...] = zero
    squ_v[...] = zero

    pltpu.sync_copy(users_hbm.at[pl.ds(base, BPW)], uidx_v)
    pltpu.sync_copy(items_hbm.at[pl.ds(wid * NCHUNK, NCHUNK), :], iidx_v)

    ucopy = pltpu.async_copy(uemb_hbm.at[uidx_v], urows_v, sem_u)

    def start_gather(g, buf, sem):
        pltpu.async_copy(iemb_hbm.at[iidx_v.at[g]], buf, sem)

    def wait_gather(buf, sem):
        pltpu.make_async_copy(
            iemb_hbm.at[iidx_v.at[0]], buf, sem).wait()

    lane15 = lax.iota(jnp.int32, 16) == 15

    def tree8(t):
        return ((t[0] + t[1]) + (t[2] + t[3])) + ((t[4] + t[5]) + (t[6] + t[7]))

    def compute_chunk(g, carry, buf):
        for j in range(CHUNK_U):
            b = g * CHUNK_U + j
            pred_base = b * L
            u = [urows_v[b, pl.ds(c * 16, 16)] for c in range(DC)]
            plsc.addupdate(squ_v.at[...], tree8([x * x for x in u]))

            @plsc.parallel_loop(0, L, step=5, unroll=2)
            def item_body(l, j=j, u=u, pred_base=pred_base):
                sqs = []
                for t in range(5):
                    row = j * L + l + t
                    iv = [buf[row, pl.ds(c * 16, 16)] for c in range(DC)]
                    prod = u[0] * iv[0]
                    sq = iv[0] * iv[0]
                    for c in range(1, DC):
                        prod = prod + u[c] * iv[c]
                        sq = sq + iv[c] * iv[c]
                    prod_scan = plsc.cumsum(prod)
                    plsc.store_scatter(
                        predbuf_v,
                        [jnp.full((16,), pred_base + l + t, jnp.int32)],
                        prod_scan,
                        mask=lane15,
                    )
                    sqs.append(sq)
                plsc.addupdate(
                    sqi_v.at[...],
                    ((sqs[0] + sqs[1]) + (sqs[2] + sqs[3])) + sqs[4])
        return carry

    for p in range(NBUF - 1):
        start_gather(p, bufs[p], sems[p])
    ucopy.wait()

    def pipe_body(k, carry):
        for p in range(NBUF):
            g = k * NBUF + p

            @pl.when(g + NBUF - 1 < NCHUNK)
            def _(g=g, p=p):
                start_gather(g + NBUF - 1, bufs[(p + NBUF - 1) % NBUF],
                             sems[(p + NBUF - 1) % NBUF])

            wait_gather(bufs[p], sems[p])
            carry = compute_chunk(g, carry, bufs[p])
        return carry

    lax.fori_loop(0, NCHUNK // NBUF, pipe_body, 0)

    l2buf_v[...] = sqi_v[...] + float(L) * squ_v[...]
    pltpu.sync_copy(l2buf_v, l2_hbm.at[wid])
    pltpu.sync_copy(predbuf_v, pred_hbm.at[pl.ds(base * L, BPW * L)])


@jax.jit
def _run(users, items2d, uemb, iemb):
    mesh = plsc.VectorSubcoreMesh(core_axis_name="c", subcore_axis_name="s")
    kern = functools.partial(
        pl.kernel,
        mesh=mesh,
        compiler_params=pltpu.CompilerParams(needs_layout_passes=False),
        out_type=[
            jax.ShapeDtypeStruct((B * L,), jnp.float32),
            jax.ShapeDtypeStruct((NW, 16), jnp.float32),
        ],
        scratch_types=[
            pltpu.VMEM((BPW,), jnp.int32),
            pltpu.VMEM((NCHUNK, CHUNK_I), jnp.int32),
            pltpu.VMEM((BPW, D), jnp.float32),
            pltpu.VMEM((CHUNK_I, D), jnp.float32),
            pltpu.VMEM((CHUNK_I, D), jnp.float32),
            pltpu.VMEM((CHUNK_I, D), jnp.float32),
            pltpu.VMEM((CHUNK_I, D), jnp.float32),
            pltpu.VMEM((BPW * L,), jnp.float32),
            pltpu.VMEM((16,), jnp.float32),
            pltpu.VMEM((16,), jnp.float32),
            pltpu.VMEM((16,), jnp.float32),
            pltpu.SemaphoreType.DMA,
            pltpu.SemaphoreType.DMA,
            pltpu.SemaphoreType.DMA,
            pltpu.SemaphoreType.DMA,
            pltpu.SemaphoreType.DMA,
        ],
    )(_sc_kernel)
    pred_flat, l2_part = kern(users, items2d, uemb, iemb)
    pred = pred_flat.reshape(B, L)
    l2 = L2_NORM * jnp.sum(l2_part)
    return pred, l2


def kernel(users, items, user_embedding, item_embedding):
    users = users.astype(jnp.int32)
    items2d = items.astype(jnp.int32).reshape(B * L // CHUNK_I, CHUNK_I)
    return _run(users, items2d, user_embedding, item_embedding)


# speedup vs baseline: 1.7140x; 1.7140x over previous
import functools

import jax
import jax.numpy as jnp
from jax import lax
from jax.experimental import pallas as pl
from jax.experimental.pallas import tpu as pltpu
from jax.experimental.pallas import tpu_sc as plsc

L2_NORM = 0.0001

B = 4096
L = 50
D = 128
NW = 32
BPW = B // NW
CHUNK_U = 2
CHUNK_I = CHUNK_U * L
NCHUNK = BPW // CHUNK_U
DC = D // 16
NBUF = 4


def _sc_kernel(users_hbm, items_hbm, uemb_hbm, iemb_hbm,
               pred_hbm, l2_hbm,
               uidx_v, iidx_v, urows_v,
               irows0_v, irows1_v, irows2_v, irows3_v, predbuf_v,
               l2buf_v, sqi_v, squ_v, sem0, sem1, sem2, sem3, sem_u):
    wid = lax.axis_index("s") * 2 + lax.axis_index("c")
    base = wid * BPW
    bufs = (irows0_v, irows1_v, irows2_v, irows3_v)
    sems = (sem0, sem1, sem2, sem3)

    zero = jnp.zeros((16,), jnp.float32)
    sqi_v[...] = zero
    squ_v[...] = zero

    pltpu.sync_copy(users_hbm.at[pl.ds(base, BPW)], uidx_v)
    pltpu.sync_copy(items_hbm.at[pl.ds(wid * NCHUNK, NCHUNK), :], iidx_v)

    ucopy = pltpu.async_copy(uemb_hbm.at[uidx_v], urows_v, sem_u)

    def start_gather(g, buf, sem):
        pltpu.async_copy(iemb_hbm.at[iidx_v.at[g]], buf, sem)

    def wait_gather(buf, sem):
        pltpu.make_async_copy(
            iemb_hbm.at[iidx_v.at[0]], buf, sem).wait()

    lane15 = lax.iota(jnp.int32, 16) == 15

    def tree8(t):
        return ((t[0] + t[1]) + (t[2] + t[3])) + ((t[4] + t[5]) + (t[6] + t[7]))

    def compute_chunk(g, carry, buf):
        for j in range(CHUNK_U):
            b = g * CHUNK_U + j
            pred_base = b * L
            u = [urows_v[b, pl.ds(c * 16, 16)] for c in range(DC)]
            plsc.addupdate(squ_v.at[...], tree8([x * x for x in u]))

            @plsc.parallel_loop(0, L, unroll=7)
            def item_body(l, j=j, u=u, pred_base=pred_base):
                row = j * L + l
                iv = [buf[row, pl.ds(c * 16, 16)] for c in range(DC)]
                prod = u[0] * iv[0]
                sq = iv[0] * iv[0]
                for c in range(1, DC):
                    prod = prod + u[c] * iv[c]
                    sq = sq + iv[c] * iv[c]
                prod_scan = plsc.cumsum(prod)
                plsc.store_scatter(
                    predbuf_v,
                    [jnp.full((16,), pred_base + l, jnp.int32)],
                    prod_scan,
                    mask=lane15,
                )
                plsc.addupdate(sqi_v.at[...], sq)
        return carry

    for p in range(NBUF - 1):
        start_gather(p, bufs[p], sems[p])
    ucopy.wait()

    def pipe_body(k, carry):
        for p in range(NBUF):
            g = k * NBUF + p

            @pl.when(g + NBUF - 1 < NCHUNK)
            def _(g=g, p=p):
                start_gather(g + NBUF - 1, bufs[(p + NBUF - 1) % NBUF],
                             sems[(p + NBUF - 1) % NBUF])

            wait_gather(bufs[p], sems[p])
            carry = compute_chunk(g, carry, bufs[p])
        return carry

    lax.fori_loop(0, NCHUNK // NBUF, pipe_body, 0)

    l2buf_v[...] = sqi_v[...] + float(L) * squ_v[...]
    pltpu.sync_copy(l2buf_v, l2_hbm.at[wid])
    pltpu.sync_copy(predbuf_v, pred_hbm.at[pl.ds(base * L, BPW * L)])


@jax.jit
def _run(users, items2d, uemb, iemb):
    mesh = plsc.VectorSubcoreMesh(core_axis_name="c", subcore_axis_name="s")
    kern = functools.partial(
        pl.kernel,
        mesh=mesh,
        compiler_params=pltpu.CompilerParams(needs_layout_passes=False),
        out_type=[
            jax.ShapeDtypeStruct((B * L,), jnp.float32),
            jax.ShapeDtypeStruct((NW, 16), jnp.float32),
        ],
        scratch_types=[
            pltpu.VMEM((BPW,), jnp.int32),
            pltpu.VMEM((NCHUNK, CHUNK_I), jnp.int32),
            pltpu.VMEM((BPW, D), jnp.float32),
            pltpu.VMEM((CHUNK_I, D), jnp.float32),
            pltpu.VMEM((CHUNK_I, D), jnp.float32),
            pltpu.VMEM((CHUNK_I, D), jnp.float32),
            pltpu.VMEM((CHUNK_I, D), jnp.float32),
            pltpu.VMEM((BPW * L,), jnp.float32),
            pltpu.VMEM((16,), jnp.float32),
            pltpu.VMEM((16,), jnp.float32),
            pltpu.VMEM((16,), jnp.float32),
            pltpu.SemaphoreType.DMA,
            pltpu.SemaphoreType.DMA,
            pltpu.SemaphoreType.DMA,
            pltpu.SemaphoreType.DMA,
            pltpu.SemaphoreType.DMA,
        ],
    )(_sc_kernel)
    pred_flat, l2_part = kern(users, items2d, uemb, iemb)
    pred = pred_flat.reshape(B, L)
    l2 = L2_NORM * jnp.sum(l2_part)
    return pred, l2


def kernel(users, items, user_embedding, item_embedding):
    users = users.astype(jnp.int32)
    items2d = items.astype(jnp.int32).reshape(B * L // CHUNK_I, CHUNK_I)
    return _run(users, items2d, user_embedding, item_embedding)


# speedup vs baseline: 2.1341x; 1.2451x over previous
import functools

import jax
import jax.numpy as jnp
from jax import lax
from jax.experimental import pallas as pl
from jax.experimental.pallas import tpu as pltpu
from jax.experimental.pallas import tpu_sc as plsc

L2_NORM = 0.0001

B = 4096
L = 50
D = 128
NW = 32
BPW = B // NW
CHUNK_U = 2
CHUNK_I = CHUNK_U * L
NCHUNK = BPW // CHUNK_U
DC = D // 16
NBUF = 4


def _sc_kernel(users_hbm, items_hbm, uemb_hbm, iemb_hbm,
               pred_hbm, l2_hbm,
               uidx_v, iidx_v, urows_v,
               irows0_v, irows1_v, irows2_v, irows3_v, predbuf_v,
               l2buf_v, sqi_v, squ_v, sem0, sem1, sem2, sem3, sem_u):
    wid = lax.axis_index("s") * 2 + lax.axis_index("c")
    base = wid * BPW
    bufs = (irows0_v, irows1_v, irows2_v, irows3_v)
    sems = (sem0, sem1, sem2, sem3)

    zero = jnp.zeros((16,), jnp.float32)
    sqi_v[...] = zero
    squ_v[...] = zero

    pltpu.sync_copy(users_hbm.at[pl.ds(base, BPW)], uidx_v)
    pltpu.sync_copy(items_hbm.at[pl.ds(wid * NCHUNK, NCHUNK), :], iidx_v)

    ucopy = pltpu.async_copy(uemb_hbm.at[uidx_v], urows_v, sem_u)

    def start_gather(g, buf, sem):
        pltpu.async_copy(iemb_hbm.at[iidx_v.at[g]], buf, sem)

    def wait_gather(buf, sem):
        pltpu.make_async_copy(
            iemb_hbm.at[iidx_v.at[0]], buf, sem).wait()

    lane15 = lax.iota(jnp.int32, 16) == 15

    def tree8(t):
        return ((t[0] + t[1]) + (t[2] + t[3])) + ((t[4] + t[5]) + (t[6] + t[7]))

    def compute_chunk(g, carry, buf):
        for j in range(CHUNK_U):
            b = g * CHUNK_U + j
            pred_base = b * L
            u = [urows_v[b, pl.ds(c * 16, 16)] for c in range(DC)]
            plsc.addupdate(squ_v.at[...], tree8([x * x for x in u]))

            @plsc.parallel_loop(0, L, unroll=4)
            def item_body(l, j=j, u=u, pred_base=pred_base):
                row = j * L + l
                iv = [buf[row, pl.ds(c * 16, 16)] for c in range(DC)]
                prod = u[0] * iv[0]
                sq = iv[0] * iv[0]
                for c in range(1, DC):
                    prod = prod + u[c] * iv[c]
                    sq = sq + iv[c] * iv[c]
                prod_scan = plsc.cumsum(prod)
                plsc.store_scatter(
                    predbuf_v,
                    [jnp.full((16,), pred_base + l, jnp.int32)],
                    prod_scan,
                    mask=lane15,
                )
                plsc.addupdate(sqi_v.at[...], sq)
        return carry

    for p in range(NBUF - 1):
        start_gather(p, bufs[p], sems[p])
    ucopy.wait()

    def pipe_body(k, carry):
        for p in range(NBUF):
            g = k * NBUF + p

            @pl.when(g + NBUF - 1 < NCHUNK)
            def _(g=g, p=p):
                start_gather(g + NBUF - 1, bufs[(p + NBUF - 1) % NBUF],
                             sems[(p + NBUF - 1) % NBUF])

            wait_gather(bufs[p], sems[p])
            carry = compute_chunk(g, carry, bufs[p])
        return carry

    lax.fori_loop(0, NCHUNK // NBUF, pipe_body, 0)

    l2buf_v[...] = sqi_v[...] + float(L) * squ_v[...]
    pltpu.sync_copy(l2buf_v, l2_hbm.at[wid])
    pltpu.sync_copy(predbuf_v, pred_hbm.at[pl.ds(base * L, BPW * L)])


@jax.jit
def _run(users, items2d, uemb, iemb):
    mesh = plsc.VectorSubcoreMesh(core_axis_name="c", subcore_axis_name="s")
    kern = functools.partial(
        pl.kernel,
        mesh=mesh,
        compiler_params=pltpu.CompilerParams(needs_layout_passes=False),
        out_type=[
            jax.ShapeDtypeStruct((B * L,), jnp.float32),
            jax.ShapeDtypeStruct((NW, 16), jnp.float32),
        ],
        scratch_types=[
            pltpu.VMEM((BPW,), jnp.int32),
            pltpu.VMEM((NCHUNK, CHUNK_I), jnp.int32),
            pltpu.VMEM((BPW, D), jnp.float32),
            pltpu.VMEM((CHUNK_I, D), jnp.float32),
            pltpu.VMEM((CHUNK_I, D), jnp.float32),
            pltpu.VMEM((CHUNK_I, D), jnp.float32),
            pltpu.VMEM((CHUNK_I, D), jnp.float32),
            pltpu.VMEM((BPW * L,), jnp.float32),
            pltpu.VMEM((16,), jnp.float32),
            pltpu.VMEM((16,), jnp.float32),
            pltpu.VMEM((16,), jnp.float32),
            pltpu.SemaphoreType.DMA,
            pltpu.SemaphoreType.DMA,
            pltpu.SemaphoreType.DMA,
            pltpu.SemaphoreType.DMA,
            pltpu.SemaphoreType.DMA,
        ],
    )(_sc_kernel)
    pred_flat, l2_part = kern(users, items2d, uemb, iemb)
    pred = pred_flat.reshape(B, L)
    l2 = L2_NORM * jnp.sum(l2_part)
    return pred, l2


def kernel(users, items, user_embedding, item_embedding):
    users = users.astype(jnp.int32)
    items2d = items.astype(jnp.int32).reshape(B * L // CHUNK_I, CHUNK_I)
    return _run(users, items2d, user_embedding, item_embedding)


# speedup vs baseline: 2.2712x; 1.0643x over previous
import functools

import jax
import jax.numpy as jnp
from jax import lax
from jax.experimental import pallas as pl
from jax.experimental.pallas import tpu as pltpu
from jax.experimental.pallas import tpu_sc as plsc

L2_NORM = 0.0001

B = 4096
L = 50
D = 128
NW = 32
BPW = B // NW
CHUNK_U = 2
CHUNK_I = CHUNK_U * L
NCHUNK = BPW // CHUNK_U
DC = D // 16
NBUF = 4


def _sc_kernel(users_hbm, items_hbm, uemb_hbm, iemb_hbm,
               pred_hbm, l2_hbm,
               uidx_v, iidx_v, urows_v,
               irows0_v, irows1_v, irows2_v, irows3_v, predbuf_v,
               l2buf_v, sqi_v, squ_v, sem0, sem1, sem2, sem3, sem_u):
    wid = lax.axis_index("s") * 2 + lax.axis_index("c")
    base = wid * BPW
    bufs = (irows0_v, irows1_v, irows2_v, irows3_v)
    sems = (sem0, sem1, sem2, sem3)

    zero = jnp.zeros((16,), jnp.float32)
    sqi_v[...] = zero
    squ_v[...] = zero

    pltpu.sync_copy(users_hbm.at[pl.ds(base, BPW)], uidx_v)
    pltpu.sync_copy(items_hbm.at[pl.ds(wid * NCHUNK, NCHUNK), :], iidx_v)

    ucopy = pltpu.async_copy(uemb_hbm.at[uidx_v], urows_v, sem_u)

    def start_gather(g, buf, sem):
        pltpu.async_copy(iemb_hbm.at[iidx_v.at[g]], buf, sem)

    def wait_gather(buf, sem):
        pltpu.make_async_copy(
            iemb_hbm.at[iidx_v.at[0]], buf, sem).wait()

    lane15 = lax.iota(jnp.int32, 16) == 15

    def tree8(t):
        return ((t[0] + t[1]) + (t[2] + t[3])) + ((t[4] + t[5]) + (t[6] + t[7]))

    def compute_chunk(g, carry, buf):
        for j in range(CHUNK_U):
            b = g * CHUNK_U + j
            pred_base = b * L
            u = [urows_v[b, pl.ds(c * 16, 16)] for c in range(DC)]
            plsc.addupdate(squ_v.at[...], tree8([x * x for x in u]))

            @plsc.parallel_loop(0, L, unroll=2)
            def item_body(l, j=j, u=u, pred_base=pred_base):
                row = j * L + l
                iv = [buf[row, pl.ds(c * 16, 16)] for c in range(DC)]
                prod = u[0] * iv[0]
                sq = iv[0] * iv[0]
                for c in range(1, DC):
                    prod = prod + u[c] * iv[c]
                    sq = sq + iv[c] * iv[c]
                prod_scan = plsc.cumsum(prod)
                plsc.store_scatter(
                    predbuf_v,
                    [jnp.full((16,), pred_base + l, jnp.int32)],
                    prod_scan,
                    mask=lane15,
                )
                plsc.addupdate(sqi_v.at[...], sq)
        return carry

    for p in range(NBUF - 1):
        start_gather(p, bufs[p], sems[p])
    ucopy.wait()

    def pipe_body(k, carry):
        for p in range(NBUF):
            g = k * NBUF + p

            @pl.when(g + NBUF - 1 < NCHUNK)
            def _(g=g, p=p):
                start_gather(g + NBUF - 1, bufs[(p + NBUF - 1) % NBUF],
                             sems[(p + NBUF - 1) % NBUF])

            wait_gather(bufs[p], sems[p])
            carry = compute_chunk(g, carry, bufs[p])
        return carry

    lax.fori_loop(0, NCHUNK // NBUF, pipe_body, 0)

    l2buf_v[...] = sqi_v[...] + float(L) * squ_v[...]
    pltpu.sync_copy(l2buf_v, l2_hbm.at[wid])
    pltpu.sync_copy(predbuf_v, pred_hbm.at[pl.ds(base * L, BPW * L)])


@jax.jit
def _run(users, items2d, uemb, iemb):
    mesh = plsc.VectorSubcoreMesh(core_axis_name="c", subcore_axis_name="s")
    kern = functools.partial(
        pl.kernel,
        mesh=mesh,
        compiler_params=pltpu.CompilerParams(needs_layout_passes=False),
        out_type=[
            jax.ShapeDtypeStruct((B * L,), jnp.float32),
            jax.ShapeDtypeStruct((NW, 16), jnp.float32),
        ],
        scratch_types=[
            pltpu.VMEM((BPW,), jnp.int32),
            pltpu.VMEM((NCHUNK, CHUNK_I), jnp.int32),
            pltpu.VMEM((BPW, D), jnp.float32),
            pltpu.VMEM((CHUNK_I, D), jnp.float32),
            pltpu.VMEM((CHUNK_I, D), jnp.float32),
            pltpu.VMEM((CHUNK_I, D), jnp.float32),
            pltpu.VMEM((CHUNK_I, D), jnp.float32),
            pltpu.VMEM((BPW * L,), jnp.float32),
            pltpu.VMEM((16,), jnp.float32),
            pltpu.VMEM((16,), jnp.float32),
            pltpu.VMEM((16,), jnp.float32),
            pltpu.SemaphoreType.DMA,
            pltpu.SemaphoreType.DMA,
            pltpu.SemaphoreType.DMA,
            pltpu.SemaphoreType.DMA,
            pltpu.SemaphoreType.DMA,
        ],
    )(_sc_kernel)
    pred_flat, l2_part = kern(users, items2d, uemb, iemb)
    pred = pred_flat.reshape(B, L)
    l2 = L2_NORM * jnp.sum(l2_part)
    return pred, l2


def kernel(users, items, user_embedding, item_embedding):
    users = users.astype(jnp.int32)
    items2d = items.astype(jnp.int32).reshape(B * L // CHUNK_I, CHUNK_I)
    return _run(users, items2d, user_embedding, item_embedding)
